# baseline (speedup 1.0000x reference)
import functools

import numpy as np
import jax
import jax.numpy as jnp
from jax import lax
from jax.experimental import pallas as pl
from jax.experimental.pallas import tpu as pltpu
from jax.experimental.pallas import tpu_sc as plsc

_SIGMA = 0.2
_B = 64
_ROW = 56 * 56 * 192
_NC, _NS = 2, 16
_NW = _NC * _NS
_W = _ROW // _NW
_NV = _W // 16

_PERM = (17, 27, 42, 32, 1, 3, 58, 51, 40, 28, 52, 19, 9, 33, 11, 45,
         31, 5, 15, 39, 50, 47, 20, 0, 46, 14, 49, 44, 38, 61, 2, 54,
         36, 35, 62, 63, 21, 59, 30, 43, 22, 18, 24, 26, 53, 12, 16, 6,
         7, 57, 55, 48, 13, 37, 60, 10, 29, 34, 25, 56, 4, 41, 23, 8)

_cache = {}


def _coeffs():
    key = jax.random.key(42)
    _, k_r, k_theta = jax.random.split(key, 3)
    rs = (1, 56, 56, 192)
    r = jax.random.normal(k_r, rs, dtype=jnp.float16) * jnp.float16(_SIGMA)
    theta = jax.random.uniform(k_theta, rs, dtype=jnp.float16,
                               minval=-np.pi, maxval=np.pi)
    a = (jnp.float16(1.0) + r * jnp.cos(theta)).astype(jnp.float32).reshape(_ROW)
    b = (r * jnp.sin(theta)).astype(jnp.float32).reshape(_ROW)
    return a, b


def _build():
    mesh = plsc.VectorSubcoreMesh(core_axis_name="c", subcore_axis_name="s")

    @functools.partial(
        pl.kernel,
        mesh=mesh,
        out_type=jax.ShapeDtypeStruct((_B * _ROW,), jnp.float32),
        scratch_types=[
            pltpu.VMEM((_W,), jnp.float32),
            pltpu.VMEM((_W,), jnp.float32),
            pltpu.VMEM((_W,), jnp.float32),
            pltpu.VMEM((_W,), jnp.float32),
            pltpu.VMEM((_W,), jnp.float32),
        ],
    )
    def mixfeat(x_hbm, a_hbm, b_hbm, y_hbm, xa_v, xb_v, a_v, b_v, o_v):
        wid = lax.axis_index("s") * _NC + lax.axis_index("c")
        base = wid * _W
        pltpu.sync_copy(a_hbm.at[pl.ds(base, _W)], a_v)
        pltpu.sync_copy(b_hbm.at[pl.ds(base, _W)], b_v)

        def compute(v, _):
            s = pl.ds(v * 16, 16)
            o_v[s] = xa_v[s] * a_v[s] + xb_v[s] * b_v[s]
            return _

        for i in range(_B):
            pltpu.sync_copy(x_hbm.at[pl.ds(i * _ROW + base, _W)], xa_v)
            pltpu.sync_copy(x_hbm.at[pl.ds(_PERM[i] * _ROW + base, _W)], xb_v)
            lax.fori_loop(0, _NV, compute, None)
            pltpu.sync_copy(o_v, y_hbm.at[pl.ds(i * _ROW + base, _W)])

    return mixfeat


def kernel(inputs):
    if "f" not in _cache:
        _cache["f"] = _build()
    a, b = _coeffs()
    x = inputs.reshape(_B * _ROW)
    y = _cache["f"](x, a, b)
    return y.reshape(inputs.shape)
